# baseline (speedup 1.0000x reference)
import functools

import jax
import jax.numpy as jnp
from jax import lax
from jax.experimental import pallas as pl
from jax.experimental.pallas import tpu as pltpu
from jax.experimental.pallas import tpu_sc as plsc

D = 32
BW = 512
HB = 256


@functools.lru_cache(maxsize=None)
def _build(batch: int, hist: int):
  info = plsc.get_sparse_core_info()
  nw = info.num_cores * info.num_subcores
  assert batch // nw == BW
  n_units = hist * 2
  mesh = plsc.VectorSubcoreMesh(core_axis_name="c", subcore_axis_name="s")

  @functools.partial(
      pl.kernel,
      mesh=mesh,
      compiler_params=pltpu.CompilerParams(needs_layout_passes=False),
      out_type=jax.ShapeDtypeStruct((hist, D, batch), jnp.float32),
      scratch_types=[
          pltpu.VMEM((hist, BW), jnp.int32),
          pltpu.VMEM((HB,), jnp.int32),
          pltpu.VMEM((HB,), jnp.int32),
          pltpu.VMEM((HB,), jnp.int32),
          pltpu.VMEM((HB,), jnp.int32),
          pltpu.VMEM((HB, 128), jnp.float32),
          pltpu.VMEM((HB, 128), jnp.float32),
          pltpu.VMEM((D, HB), jnp.float32),
          pltpu.VMEM((D, HB), jnp.float32),
          pltpu.SemaphoreType.DMA,
          pltpu.SemaphoreType.DMA,
      ],
  )
  def gather_kernel(table_hbm, idxt_hbm, out_hbm, idx_v, sup0, sup1, mod0,
                    mod1, rows0, rows1, obuf0, obuf1, sem_g, sem_o):
    wid = lax.axis_index("s") * info.num_cores + lax.axis_index("c")
    b0 = wid * BW
    lane = lax.iota(jnp.int32, 16)
    sups = (sup0, sup1)
    mods = (mod0, mod1)
    rows = (rows0, rows1)
    obufs = (obuf0, obuf1)
    pltpu.sync_copy(idxt_hbm.at[:, pl.ds(b0, BW)], idx_v)

    def split(u, slot):
      h = u >> 1
      off = (u & 1) * HB

      def body(k, carry):
        v = idx_v[h, pl.ds(off + k * 16, 16)]
        sups[slot][pl.ds(k * 16, 16)] = v >> 2
        mods[slot][pl.ds(k * 16, 16)] = (v & 3) * 32
        return carry

      lax.fori_loop(0, HB // 16, body, 0, unroll=4)

    def start_gather(u, slot):
      return pltpu.async_copy(table_hbm.at[sups[slot]], rows[slot], sem_g)

    def extract(u, slot):
      def bg_body(bg, carry):
        row_ids = bg * 16 + lane
        mod_slice = mods[slot][pl.ds(bg * 16, 16)]

        def d_body(d, carry2):
          obufs[slot][d, pl.ds(bg * 16, 16)] = plsc.load_gather(
              rows[slot], [row_ids, mod_slice + d]
          )
          return carry2

        lax.fori_loop(0, D, d_body, 0, unroll=16)
        return carry

      lax.fori_loop(0, HB // 16, bg_body, 0, unroll=2)

    def start_out(u, slot):
      h = u >> 1
      boff = b0 + (u & 1) * HB
      return pltpu.async_copy(
          obufs[slot], out_hbm.at[h, :, pl.ds(boff, HB)], sem_o
      )

    def wait_gather(slot):
      pltpu.make_async_copy(
          table_hbm.at[sups[slot]], rows[slot], sem_g
      ).wait()

    def wait_out(slot):
      pltpu.make_async_copy(
          obufs[slot], out_hbm.at[0, :, pl.ds(b0, HB)], sem_o
      ).wait()

    split(0, 0)
    start_gather(0, 0)
    split(1, 1)
    start_gather(1, 1)

    def pair_body(p, carry):
      for slot in (0, 1):
        u = 2 * p + slot
        wait_gather(slot)

        @pl.when(u >= 2)
        def _():
          wait_out(slot)

        extract(u, slot)
        start_out(u, slot)

        @pl.when(u + 2 < n_units)
        def _():
          split(u + 2, slot)
          start_gather(u + 2, slot)

      return carry

    lax.fori_loop(0, n_units // 2, pair_body, 0)
    wait_out(0)
    wait_out(1)

  return gather_kernel


def kernel(idx, table):
  b, h = idx.shape
  idx_t = idx.T.astype(jnp.int32)
  table2 = table.reshape(250000, 128)
  out3 = _build(b, h)(table2, idx_t)
  return out3.transpose(2, 0, 1)


# speedup vs baseline: 1.3086x; 1.3086x over previous
import functools

import jax
import jax.numpy as jnp
from jax import lax
from jax.experimental import pallas as pl
from jax.experimental.pallas import tpu as pltpu
from jax.experimental.pallas import tpu_sc as plsc

D = 32
BW = 512
HB = 256


@functools.lru_cache(maxsize=None)
def _build(batch: int, hist: int):
  info = plsc.get_sparse_core_info()
  nw = info.num_cores * info.num_subcores
  assert batch // nw == BW
  n_units = hist * 2
  mesh = plsc.VectorSubcoreMesh(core_axis_name="c", subcore_axis_name="s")

  @functools.partial(
      pl.kernel,
      mesh=mesh,
      compiler_params=pltpu.CompilerParams(needs_layout_passes=False),
      out_type=jax.ShapeDtypeStruct((hist, D, batch), jnp.float32),
      scratch_types=[
          pltpu.VMEM((hist, BW), jnp.int32),
          pltpu.VMEM((HB,), jnp.int32),
          pltpu.VMEM((HB,), jnp.int32),
          pltpu.VMEM((HB,), jnp.int32),
          pltpu.VMEM((HB,), jnp.int32),
          pltpu.VMEM((HB, 128), jnp.float32),
          pltpu.VMEM((HB, 128), jnp.float32),
          pltpu.VMEM((D, HB), jnp.float32),
          pltpu.VMEM((D, HB), jnp.float32),
          pltpu.SemaphoreType.DMA,
          pltpu.SemaphoreType.DMA,
      ],
  )
  def gather_kernel(table_hbm, idxt_hbm, out_hbm, idx_v, sup0, sup1, mod0,
                    mod1, rows0, rows1, obuf0, obuf1, sem_g, sem_o):
    wid = lax.axis_index("s") * info.num_cores + lax.axis_index("c")
    b0 = wid * BW
    lane = lax.iota(jnp.int32, 16)
    sups = (sup0, sup1)
    mods = (mod0, mod1)
    rows = (rows0, rows1)
    obufs = (obuf0, obuf1)
    pltpu.sync_copy(idxt_hbm.at[:, pl.ds(b0, BW)], idx_v)

    def split(u, slot):
      h = u >> 1
      off = (u & 1) * HB

      def body(k, carry):
        v = idx_v[h, pl.ds(off + k * 16, 16)]
        sups[slot][pl.ds(k * 16, 16)] = v >> 2
        mods[slot][pl.ds(k * 16, 16)] = (v & 3) * 32
        return carry

      lax.fori_loop(0, HB // 16, body, 0, unroll=4)

    def start_gather(u, slot):
      return pltpu.async_copy(
          table_hbm.at[sups[slot]], rows[slot], sem_g
      )

    def extract(u, slot):
      @plsc.parallel_loop(0, HB // 16)
      def bg_body(bg):
        row_ids = bg * 16 + lane
        mod_slice = mods[slot][pl.ds(bg * 16, 16)]

        @plsc.parallel_loop(0, D, unroll=8)
        def d_body(d):
          obufs[slot][d, pl.ds(bg * 16, 16)] = plsc.load_gather(
              rows[slot], [row_ids, mod_slice + d]
          )

    def start_out(u, slot):
      h = u >> 1
      boff = b0 + (u & 1) * HB
      return pltpu.async_copy(
          obufs[slot], out_hbm.at[h, :, pl.ds(boff, HB)], sem_o
      )

    def wait_gather(slot):
      pltpu.make_async_copy(
          table_hbm.at[sups[slot]], rows[slot], sem_g
      ).wait()

    def wait_out(slot):
      pltpu.make_async_copy(
          obufs[slot], out_hbm.at[0, :, pl.ds(b0, HB)], sem_o
      ).wait()

    split(0, 0)
    start_gather(0, 0)
    split(1, 1)
    start_gather(1, 1)

    def pair_body(p, carry):
      for slot in (0, 1):
        u = 2 * p + slot
        wait_gather(slot)

        @pl.when(u >= 2)
        def _():
          wait_out(slot)

        extract(u, slot)
        start_out(u, slot)

        @pl.when(u + 2 < n_units)
        def _():
          split(u + 2, slot)
          start_gather(u + 2, slot)

      return carry

    lax.fori_loop(0, n_units // 2, pair_body, 0)
    wait_out(0)
    wait_out(1)

  return gather_kernel


def kernel(idx, table):
  b, h = idx.shape
  idx_t = idx.T.astype(jnp.int32)
  table2 = table.reshape(250000, 128)
  out3 = _build(b, h)(table2, idx_t)
  return out3.transpose(2, 0, 1)
